# Initial kernel scaffold; baseline (speedup 1.0000x reference)
import jax
import jax.numpy as jnp
from jax.experimental import pallas as pl


def kernel(x, edge_index, W, b):
    raise NotImplementedError("write your pallas kernel here")


# speedup vs baseline: 3.3564x; 3.3564x over previous
import functools

import jax
import jax.numpy as jnp
from jax import lax
from jax.experimental import pallas as pl
from jax.experimental.pallas import tpu as pltpu
from jax.experimental.pallas import tpu_sc as plsc

N_NODES = 10000
N_EDGES = 320000
D = 128

NC = 2
NS = 16
NW = NC * NS

CHUNK = 128
CHUNKS_T = 80
IBLK = 16
NBLK = CHUNKS_T // IBLK
E_PAD = NW * CHUNKS_T * CHUNK
N_PAD = 10112
ZROWS = N_PAD // NS


def _sc_aggregate(src2d, dst2d, x):
    mesh = plsc.VectorSubcoreMesh(core_axis_name="c", subcore_axis_name="s")

    @functools.partial(
        pl.kernel,
        out_type=jax.ShapeDtypeStruct((NC * N_PAD, D), jnp.float32),
        mesh=mesh,
        scratch_types=[
            pltpu.VMEM((2, IBLK, CHUNK), jnp.int32),
            pltpu.VMEM((2, IBLK, CHUNK), jnp.int32),
            pltpu.VMEM((CHUNK, D), jnp.float32),
            pltpu.VMEM((CHUNK, D), jnp.float32),
            pltpu.VMEM_SHARED((N_PAD, D), jnp.float32),
            pltpu.SemaphoreType.DMA,
            pltpu.SemaphoreType.DMA,
            pltpu.SemaphoreType.DMA,
        ],
    )
    def sc_kernel(src_hbm, dst_hbm, x_hbm, out_hbm,
                  src_v, dst_v, bufa, bufb, agg, sema, semb, semi):
        cid = lax.axis_index("c")
        sid = lax.axis_index("s")
        tid = cid * NS + sid

        @pl.loop(0, CHUNK)
        def _(i):
            for k in range(D // 16):
                bufa[i, pl.ds(k * 16, 16)] = jnp.zeros((16,), jnp.float32)

        zbase = sid * ZROWS
        nfull = ZROWS // CHUNK
        for z in range(nfull):
            pltpu.sync_copy(bufa, agg.at[pl.ds(zbase + z * CHUNK, CHUNK)])
        rem = ZROWS - nfull * CHUNK
        if rem:
            pltpu.sync_copy(bufa.at[pl.ds(0, rem)],
                            agg.at[pl.ds(zbase + nfull * CHUNK, rem)])
        plsc.subcore_barrier()

        base = tid * CHUNKS_T
        pltpu.sync_copy(src_hbm.at[pl.ds(base, IBLK)], src_v.at[0])
        pltpu.sync_copy(dst_hbm.at[pl.ds(base, IBLK)], dst_v.at[0])

        for blk in range(NBLK):
            cur = blk % 2
            nxt = 1 - cur
            if blk + 1 < NBLK:
                hs = pltpu.async_copy(
                    src_hbm.at[pl.ds(base + (blk + 1) * IBLK, IBLK)],
                    src_v.at[nxt], semi)
                hd = pltpu.async_copy(
                    dst_hbm.at[pl.ds(base + (blk + 1) * IBLK, IBLK)],
                    dst_v.at[nxt], semi)
            sv = src_v.at[cur]
            dv = dst_v.at[cur]
            pltpu.async_copy(x_hbm.at[sv.at[0]], bufa, sema)

            @pl.loop(0, IBLK // 2)
            def _(g):
                j0 = g * 2
                j1 = j0 + 1
                pltpu.async_copy(x_hbm.at[sv.at[j1]], bufb, semb)
                pltpu.make_async_copy(x_hbm.at[sv.at[j0]], bufa, sema).wait()
                pltpu.sync_copy(bufa, agg.at[dv.at[j0]], add=True)

                @pl.when(j1 + 1 < IBLK)
                def _():
                    pltpu.async_copy(x_hbm.at[sv.at[j1 + 1]], bufa, sema)

                pltpu.make_async_copy(x_hbm.at[sv.at[j1]], bufb, semb).wait()
                pltpu.sync_copy(bufb, agg.at[dv.at[j1]], add=True)

            if blk + 1 < NBLK:
                hs.wait()
                hd.wait()

        plsc.subcore_barrier()

        obase = sid * ZROWS
        pltpu.sync_copy(agg.at[pl.ds(obase, ZROWS)],
                        out_hbm.at[pl.ds(cid * N_PAD + obase, ZROWS)])

    return sc_kernel(src2d, dst2d, x)


def _tc_body(x_ref, a0_ref, a1_ref, w_ref, b_ref, o_ref):
    h = x_ref[...] + a0_ref[0] + a1_ref[0]
    o_ref[...] = lax.dot_general(
        h, w_ref[...],
        dimension_numbers=(((1,), (1,)), ((), ())),
        preferred_element_type=jnp.float32,
    ) + b_ref[...]


def kernel(x, edge_index, W, b):
    src = edge_index[0]
    dst = edge_index[1]
    pad = E_PAD - N_EDGES
    src_p = jnp.concatenate([src, jnp.zeros((pad,), jnp.int32)])
    dst_p = jnp.concatenate([dst, jnp.full((pad,), N_NODES, jnp.int32)])
    src2d = src_p.reshape(NW * CHUNKS_T, CHUNK)
    dst2d = dst_p.reshape(NW * CHUNKS_T, CHUNK)

    agg = _sc_aggregate(src2d, dst2d, x).reshape(NC, N_PAD, D)

    BM = 1000
    nb = N_NODES // BM
    out = pl.pallas_call(
        _tc_body,
        grid=(nb,),
        in_specs=[
            pl.BlockSpec((BM, D), lambda i: (i, 0)),
            pl.BlockSpec((1, BM, D), lambda i: (0, i, 0)),
            pl.BlockSpec((1, BM, D), lambda i: (1, i, 0)),
            pl.BlockSpec((D, D), lambda i: (0, 0)),
            pl.BlockSpec((1, D), lambda i: (0, 0)),
        ],
        out_specs=pl.BlockSpec((BM, D), lambda i: (i, 0)),
        out_shape=jax.ShapeDtypeStruct((N_NODES, D), jnp.float32),
    )(x, agg, agg, W, b.reshape(1, D))
    return out


# speedup vs baseline: 12.5387x; 3.7357x over previous
import functools

import jax
import jax.numpy as jnp
from jax import lax
from jax.experimental import pallas as pl
from jax.experimental.pallas import tpu as pltpu
from jax.experimental.pallas import tpu_sc as plsc

N_NODES = 10000
N_EDGES = 320000
D = 128

NC = 2
NS = 16
NW = NC * NS

CHUNK = 128
CHUNKS_T = 80
IBLK = 16
NBLK = CHUNKS_T // IBLK
E_PAD = NW * CHUNKS_T * CHUNK
N_PAD = 10112
ZROWS = N_PAD // NS


def _sc_aggregate(src2d, dst2d, x):
    mesh = plsc.VectorSubcoreMesh(core_axis_name="c", subcore_axis_name="s")

    @functools.partial(
        pl.kernel,
        out_type=jax.ShapeDtypeStruct((NC * N_PAD, D), jnp.float32),
        mesh=mesh,
        scratch_types=[
            pltpu.VMEM((2, IBLK, CHUNK), jnp.int32),
            pltpu.VMEM((2, IBLK, CHUNK), jnp.int32),
            pltpu.VMEM((CHUNK, D), jnp.float32),
            pltpu.VMEM((CHUNK, D), jnp.float32),
            pltpu.VMEM_SHARED((N_PAD, D), jnp.float32),
            pltpu.SemaphoreType.DMA,
            pltpu.SemaphoreType.DMA,
            pltpu.SemaphoreType.DMA,
        ],
    )
    def sc_kernel(src_hbm, dst_hbm, x_hbm, out_hbm,
                  src_v, dst_v, bufa, bufb, agg, sema, semb, semi):
        cid = lax.axis_index("c")
        sid = lax.axis_index("s")
        tid = cid * NS + sid

        @pl.loop(0, CHUNK)
        def _(i):
            for k in range(D // 16):
                bufa[i, pl.ds(k * 16, 16)] = jnp.zeros((16,), jnp.float32)

        zbase = sid * ZROWS
        nfull = ZROWS // CHUNK
        for z in range(nfull):
            pltpu.sync_copy(bufa, agg.at[pl.ds(zbase + z * CHUNK, CHUNK)])
        rem = ZROWS - nfull * CHUNK
        if rem:
            pltpu.sync_copy(bufa.at[pl.ds(0, rem)],
                            agg.at[pl.ds(zbase + nfull * CHUNK, rem)])
        plsc.subcore_barrier()

        base = tid * CHUNKS_T
        pltpu.sync_copy(src_hbm.at[pl.ds(base, IBLK)], src_v.at[0])
        pltpu.sync_copy(dst_hbm.at[pl.ds(base, IBLK)], dst_v.at[0])

        for blk in range(NBLK):
            cur = blk % 2
            nxt = 1 - cur
            if blk + 1 < NBLK:
                hs = pltpu.async_copy(
                    src_hbm.at[pl.ds(base + (blk + 1) * IBLK, IBLK)],
                    src_v.at[nxt], semi)
                hd = pltpu.async_copy(
                    dst_hbm.at[pl.ds(base + (blk + 1) * IBLK, IBLK)],
                    dst_v.at[nxt], semi)
            sv = src_v.at[cur]
            dv = dst_v.at[cur]
            pltpu.async_copy(x_hbm.at[sv.at[0]], bufa, sema)

            @pl.loop(0, IBLK // 2)
            def _(g):
                j0 = g * 2
                j1 = j0 + 1
                pltpu.async_copy(x_hbm.at[sv.at[j1]], bufb, semb)
                pltpu.make_async_copy(x_hbm.at[sv.at[j0]], bufa, sema).wait()
                pltpu.sync_copy(bufa, agg.at[dv.at[j0]], add=True)

                @pl.when(j1 + 1 < IBLK)
                def _():
                    pltpu.async_copy(x_hbm.at[sv.at[j1 + 1]], bufa, sema)

                pltpu.make_async_copy(x_hbm.at[sv.at[j1]], bufb, semb).wait()
                pltpu.sync_copy(bufb, agg.at[dv.at[j1]], add=True)

            if blk + 1 < NBLK:
                hs.wait()
                hd.wait()

        plsc.subcore_barrier()

        obase = sid * ZROWS
        pltpu.sync_copy(agg.at[pl.ds(obase, ZROWS)],
                        out_hbm.at[pl.ds(cid * N_PAD + obase, ZROWS)])

    return sc_kernel(src2d, dst2d, x)


def _tc_body(x_ref, a0_ref, a1_ref, w_ref, b_ref, o_ref):
    h = x_ref[...] + a0_ref[0] + a1_ref[0]
    o_ref[...] = lax.dot_general(
        h, w_ref[...],
        dimension_numbers=(((1,), (1,)), ((), ())),
        preferred_element_type=jnp.float32,
    ) + b_ref[...]


def kernel(x, edge_index, W, b):
    src = edge_index[0]
    dst = edge_index[1]
    pad = E_PAD - N_EDGES
    pad_i = jnp.arange(pad, dtype=jnp.int32)
    src_p = jnp.concatenate([src, pad_i % N_NODES])
    dst_p = jnp.concatenate([dst, N_NODES + pad_i % (N_PAD - N_NODES)])
    src2d = src_p.reshape(NW * CHUNKS_T, CHUNK)
    dst2d = dst_p.reshape(NW * CHUNKS_T, CHUNK)

    agg = _sc_aggregate(src2d, dst2d, x).reshape(NC, N_PAD, D)

    BM = 1000
    nb = N_NODES // BM
    out = pl.pallas_call(
        _tc_body,
        grid=(nb,),
        in_specs=[
            pl.BlockSpec((BM, D), lambda i: (i, 0)),
            pl.BlockSpec((1, BM, D), lambda i: (0, i, 0)),
            pl.BlockSpec((1, BM, D), lambda i: (1, i, 0)),
            pl.BlockSpec((D, D), lambda i: (0, 0)),
            pl.BlockSpec((1, D), lambda i: (0, 0)),
        ],
        out_specs=pl.BlockSpec((BM, D), lambda i: (i, 0)),
        out_shape=jax.ShapeDtypeStruct((N_NODES, D), jnp.float32),
    )(x, agg, agg, W, b.reshape(1, D))
    return out
